# baseline (speedup 1.0000x reference)
import functools

import jax
import jax.numpy as jnp
from jax import lax
from jax.experimental import pallas as pl
from jax.experimental.pallas import tpu as pltpu
from jax.experimental.pallas import tpu_sc as plsc

N = 10000
E = 320000
D_IN = 128
D_EDGE = 16
H = 128
D_OUT = 128
U_DIM = 64
G = 16

NC, NS = 2, 16
NW = NC * NS
EPW = 10240
EPAD = NW * EPW
GPW = EPW // 128
GS = 256
NGC = EPW // GS
NGP = NGC // 2
SS = 128
NSC = EPW // SS
NSP = NSC // 2
NACC = 10240
TRASH = N
RPT = NACC // NS

_sc_mesh = plsc.VectorSubcoreMesh(core_axis_name="c", subcore_axis_name="s",
                                  num_cores=NC, num_subcores=NS)


@functools.partial(
    pl.kernel, mesh=_sc_mesh,
    out_type=jax.ShapeDtypeStruct((EPAD, D_IN), jnp.float32),
    scratch_types=[
        pltpu.VMEM((GPW, 128), jnp.int32),
        pltpu.VMEM((GS, D_IN), jnp.float32),
        pltpu.VMEM((GS, D_IN), jnp.float32),
        pltpu.SemaphoreType.DMA,
        pltpu.SemaphoreType.DMA,
    ],
)
def _sc_gather(x_hbm, col_hbm, out_hbm, idx_v, buf0, buf1, sem0, sem1):
    core = lax.axis_index("c")
    wid = lax.axis_index("s") * NC + core
    pltpu.sync_copy(col_hbm.at[pl.ds(wid * GPW, GPW)], idx_v)

    def fire(c, buf, sem):
        for j in range(2):
            pltpu.async_copy(x_hbm.at[core].at[idx_v.at[2 * c + j]],
                             buf.at[pl.ds(j * 128, 128)], sem)

    def drain(buf, sem):
        for j in range(2):
            pltpu.make_async_copy(x_hbm.at[core].at[idx_v.at[0]],
                                  buf.at[pl.ds(j * 128, 128)], sem).wait()

    def store(c, buf):
        pltpu.sync_copy(buf, out_hbm.at[pl.ds(wid * EPW + c * GS, GS)])

    fire(0, buf0, sem0)

    def pair(i, carry):
        fire(2 * i + 1, buf1, sem1)
        drain(buf0, sem0)
        store(2 * i, buf0)

        @pl.when(i < NGP - 1)
        def _():
            fire(2 * i + 2, buf0, sem0)

        drain(buf1, sem1)
        store(2 * i + 1, buf1)
        return carry

    lax.fori_loop(0, NGP, pair, 0)


@functools.partial(
    pl.kernel, mesh=_sc_mesh,
    out_type=(
        jax.ShapeDtypeStruct((NC, NACC, H), jnp.float32),
        jax.ShapeDtypeStruct((NC, NACC), jnp.float32),
    ),
    scratch_types=[
        pltpu.VMEM((GPW, 128), jnp.int32),
        pltpu.VMEM((SS, H), jnp.float32),
        pltpu.VMEM((SS, H), jnp.float32),
        pltpu.VMEM((128,), jnp.float32),
        pltpu.SemaphoreType.DMA,
        pltpu.SemaphoreType.DMA,
        pltpu.VMEM_SHARED((NACC, H), jnp.float32),
        pltpu.VMEM_SHARED((NACC,), jnp.float32),
    ],
)
def _sc_scatter(msg_hbm, row_hbm, zrows_hbm, zcnt_hbm, ones_hbm,
                sums_hbm, cnt_hbm, idx_v, buf0, buf1, ones_v, sem0, sem1,
                acc_sh, cacc_sh):
    c = lax.axis_index("c")
    s = lax.axis_index("s")
    wid = s * NC + c
    pltpu.sync_copy(zrows_hbm.at[pl.ds(s * RPT, RPT)], acc_sh.at[pl.ds(s * RPT, RPT)])
    pltpu.sync_copy(zcnt_hbm.at[pl.ds(s * RPT, RPT)], cacc_sh.at[pl.ds(s * RPT, RPT)])
    pltpu.sync_copy(ones_hbm, ones_v)
    pltpu.sync_copy(row_hbm.at[pl.ds(wid * GPW, GPW)], idx_v)
    plsc.subcore_barrier()

    def fire(ch, buf, sem):
        pltpu.async_copy(msg_hbm.at[pl.ds(wid * EPW + ch * SS, SS)], buf, sem)

    def drain(buf, sem):
        pltpu.make_async_copy(msg_hbm.at[pl.ds(0, SS)], buf, sem).wait()

    def scat(ch, buf):
        pltpu.sync_copy(buf, acc_sh.at[idx_v.at[ch]], add=True)
        pltpu.sync_copy(ones_v, cacc_sh.at[idx_v.at[ch]], add=True)

    fire(0, buf0, sem0)

    def pair(i, carry):
        fire(2 * i + 1, buf1, sem1)
        drain(buf0, sem0)
        scat(2 * i, buf0)

        @pl.when(i < NSP - 1)
        def _():
            fire(2 * i + 2, buf0, sem0)

        drain(buf1, sem1)
        scat(2 * i + 1, buf1)
        return carry

    lax.fori_loop(0, NSP, pair, 0)
    plsc.subcore_barrier()
    pltpu.sync_copy(acc_sh.at[pl.ds(s * RPT, RPT)], sums_hbm.at[c, pl.ds(s * RPT, RPT)])
    pltpu.sync_copy(cacc_sh.at[pl.ds(s * RPT, RPT)], cnt_hbm.at[c, pl.ds(s * RPT, RPT)])


BE = 2560


def _edge_mlp_body(g_ref, ea_ref, w1a_ref, b1a_ref, w1b_ref, b1b_ref, out_ref):
    g = g_ref[...].astype(jnp.float32)
    ea = ea_ref[...]
    h = jnp.dot(g, w1a_ref[0:D_IN, :], preferred_element_type=jnp.float32)
    h += jnp.dot(ea, w1a_ref[D_IN:D_IN + D_EDGE, :], preferred_element_type=jnp.float32)
    h = jax.nn.relu(h + b1a_ref[...])
    h = jnp.dot(h, w1b_ref[...], preferred_element_type=jnp.float32) + b1b_ref[...]
    out_ref[...] = jax.nn.relu(h)


def _edge_mlp(gathered, ea, W1a, b1a, W1b, b1b):
    grid = (EPAD // BE,)
    return pl.pallas_call(
        _edge_mlp_body,
        grid=grid,
        in_specs=[
            pl.BlockSpec((BE, D_IN), lambda i: (i, 0)),
            pl.BlockSpec((BE, D_EDGE), lambda i: (i, 0)),
            pl.BlockSpec((D_IN + D_EDGE, H), lambda i: (0, 0)),
            pl.BlockSpec((1, H), lambda i: (0, 0)),
            pl.BlockSpec((H, H), lambda i: (0, 0)),
            pl.BlockSpec((1, H), lambda i: (0, 0)),
        ],
        out_specs=pl.BlockSpec((BE, H), lambda i: (i, 0)),
        out_shape=jax.ShapeDtypeStruct((EPAD, H), jnp.float32),
        compiler_params=pltpu.CompilerParams(
            dimension_semantics=("arbitrary",)),
    )(gathered, ea, W1a, b1a, W1b, b1b)


BN = 2000


def _node_mlp_body(x_ref, sums_ref, cnt_ref, batch_ref, u_ref,
                   w2a_ref, b2a_ref, w2b_ref, b2b_ref, out_ref):
    x = x_ref[...]
    sums = sums_ref[0] + sums_ref[1]
    cnt = cnt_ref[0] + cnt_ref[1]
    mean = sums / jnp.maximum(cnt, 1.0)
    b = batch_ref[...]
    iota_g = lax.broadcasted_iota(jnp.int32, (1, G), 1)
    onehot = (b == iota_g).astype(jnp.float32)
    ug = jnp.dot(onehot, u_ref[...], preferred_element_type=jnp.float32)
    h = jnp.dot(x, w2a_ref[0:D_IN, :], preferred_element_type=jnp.float32)
    h += jnp.dot(mean, w2a_ref[D_IN:D_IN + H, :], preferred_element_type=jnp.float32)
    h += jnp.dot(ug, w2a_ref[D_IN + H:D_IN + H + U_DIM, :],
                 preferred_element_type=jnp.float32)
    h = jax.nn.relu(h + b2a_ref[...])
    out_ref[...] = jnp.dot(h, w2b_ref[...], preferred_element_type=jnp.float32) \
        + b2b_ref[...]


def _node_mlp(x, sums, cnt, batch2d, u, W2a, b2a, W2b, b2b):
    grid = (N // BN,)
    return pl.pallas_call(
        _node_mlp_body,
        grid=grid,
        in_specs=[
            pl.BlockSpec((BN, D_IN), lambda i: (i, 0)),
            pl.BlockSpec((NC, BN, H), lambda i: (0, i, 0)),
            pl.BlockSpec((NC, BN, 1), lambda i: (0, i, 0)),
            pl.BlockSpec((BN, 1), lambda i: (i, 0)),
            pl.BlockSpec((G, U_DIM), lambda i: (0, 0)),
            pl.BlockSpec((D_IN + H + U_DIM, H), lambda i: (0, 0)),
            pl.BlockSpec((1, H), lambda i: (0, 0)),
            pl.BlockSpec((H, D_OUT), lambda i: (0, 0)),
            pl.BlockSpec((1, D_OUT), lambda i: (0, 0)),
        ],
        out_specs=pl.BlockSpec((BN, D_OUT), lambda i: (i, 0)),
        out_shape=jax.ShapeDtypeStruct((N, D_OUT), jnp.float32),
        compiler_params=pltpu.CompilerParams(
            dimension_semantics=("arbitrary",)),
    )(x, sums, cnt, batch2d, u, W2a, b2a, W2b, b2b)


def kernel(x, edge_index, edge_attr, u, batch, W1a, b1a, W1b, b1b,
           W2a, b2a, W2b, b2b):
    row = edge_index[0]
    col = edge_index[1]
    pad = EPAD - E
    col_p = jnp.concatenate([col, jnp.zeros((pad,), jnp.int32)]).reshape(EPAD // 128, 128)
    row_p = jnp.concatenate([row, jnp.full((pad,), TRASH, jnp.int32)]).reshape(EPAD // 128, 128)
    ea_p = jnp.concatenate([edge_attr, jnp.zeros((pad, D_EDGE), jnp.float32)], axis=0)

    gathered = _sc_gather(jnp.tile(x[None], (NC, 1, 1)), col_p)
    msg = _edge_mlp(gathered, ea_p, W1a, b1a.reshape(1, H), W1b, b1b.reshape(1, H))

    zrows = jnp.zeros((NACC, H), jnp.float32)
    zcnt = jnp.zeros((NACC,), jnp.float32)
    ones = jnp.ones((128,), jnp.float32)
    sums, cnt = _sc_scatter(msg, row_p, zrows, zcnt, ones)

    out = _node_mlp(x, sums[:, :N, :], cnt[:, :N].reshape(NC, N, 1), batch.reshape(N, 1), u,
                    W2a, b2a.reshape(1, H), W2b, b2b.reshape(1, D_OUT))
    return out


# speedup vs baseline: 2.0034x; 2.0034x over previous
import functools

import jax
import jax.numpy as jnp
from jax import lax
from jax.experimental import pallas as pl
from jax.experimental.pallas import tpu as pltpu
from jax.experimental.pallas import tpu_sc as plsc

N = 10000
E = 320000
D_IN = 128
D_EDGE = 16
H = 128
D_OUT = 128
U_DIM = 64
G = 16

NC, NS = 2, 16
NW = NC * NS
EPW = 10240
EPAD = NW * EPW
GPW = EPW // 128
GS = 256
NGC = EPW // GS
NGP = NGC // 2
SS = 128
NSC = EPW // SS
NSP = NSC // 2
NACC = 10240
TRASH = N
RPT = NACC // NS

_sc_mesh = plsc.VectorSubcoreMesh(core_axis_name="c", subcore_axis_name="s",
                                  num_cores=NC, num_subcores=NS)


@functools.partial(
    pl.kernel, mesh=_sc_mesh,
    out_type=jax.ShapeDtypeStruct((EPAD, D_IN), jnp.float32),
    scratch_types=[
        pltpu.VMEM((GPW, 128), jnp.int32),
        pltpu.VMEM((128, D_IN), jnp.float32),
        pltpu.VMEM((128, D_IN), jnp.float32),
        pltpu.SemaphoreType.DMA,
        pltpu.SemaphoreType.DMA,
        pltpu.VMEM_SHARED((NACC, D_IN), jnp.float32),
    ],
)
def _sc_gather(x_hbm, col_hbm, out_hbm, idx_v, buf0, buf1, sem0, sem1, xs_sh):
    s = lax.axis_index("s")
    wid = s * NC + lax.axis_index("c")

    @pl.when(s < NS - 1)
    def _():
        pltpu.sync_copy(x_hbm.at[pl.ds(s * RPT, RPT)], xs_sh.at[pl.ds(s * RPT, RPT)])

    @pl.when(s == NS - 1)
    def _():
        pltpu.sync_copy(x_hbm.at[pl.ds((NS - 1) * RPT, N - (NS - 1) * RPT)],
                        xs_sh.at[pl.ds((NS - 1) * RPT, N - (NS - 1) * RPT)])

    pltpu.sync_copy(col_hbm.at[pl.ds(wid * GPW, GPW)], idx_v)
    plsc.subcore_barrier()

    def fire(c, buf, sem):
        pltpu.async_copy(xs_sh.at[idx_v.at[c]], buf, sem)

    def drain(buf, sem):
        pltpu.make_async_copy(xs_sh.at[idx_v.at[0]], buf, sem).wait()

    def store(c, buf):
        pltpu.sync_copy(buf, out_hbm.at[pl.ds(wid * EPW + c * 128, 128)])

    fire(0, buf0, sem0)

    def pair(i, carry):
        fire(2 * i + 1, buf1, sem1)
        drain(buf0, sem0)
        store(2 * i, buf0)

        @pl.when(i < GPW // 2 - 1)
        def _():
            fire(2 * i + 2, buf0, sem0)

        drain(buf1, sem1)
        store(2 * i + 1, buf1)
        return carry

    lax.fori_loop(0, GPW // 2, pair, 0)


@functools.partial(
    pl.kernel, mesh=_sc_mesh,
    out_type=(
        jax.ShapeDtypeStruct((NC, NACC, H), jnp.float32),
        jax.ShapeDtypeStruct((NC, NACC), jnp.float32),
    ),
    scratch_types=[
        pltpu.VMEM((GPW, 128), jnp.int32),
        pltpu.VMEM((SS, H), jnp.float32),
        pltpu.VMEM((SS, H), jnp.float32),
        pltpu.VMEM((128,), jnp.float32),
        pltpu.SemaphoreType.DMA,
        pltpu.SemaphoreType.DMA,
        pltpu.VMEM_SHARED((NACC, H), jnp.float32),
        pltpu.VMEM_SHARED((NACC,), jnp.float32),
    ],
)
def _sc_scatter(msg_hbm, row_hbm, zrows_hbm, zcnt_hbm, ones_hbm,
                sums_hbm, cnt_hbm, idx_v, buf0, buf1, ones_v, sem0, sem1,
                acc_sh, cacc_sh):
    c = lax.axis_index("c")
    s = lax.axis_index("s")
    wid = s * NC + c
    pltpu.sync_copy(zrows_hbm.at[pl.ds(s * RPT, RPT)], acc_sh.at[pl.ds(s * RPT, RPT)])
    pltpu.sync_copy(zcnt_hbm.at[pl.ds(s * RPT, RPT)], cacc_sh.at[pl.ds(s * RPT, RPT)])
    pltpu.sync_copy(ones_hbm, ones_v)
    pltpu.sync_copy(row_hbm.at[pl.ds(wid * GPW, GPW)], idx_v)
    plsc.subcore_barrier()

    def fire(ch, buf, sem):
        pltpu.async_copy(msg_hbm.at[pl.ds(wid * EPW + ch * SS, SS)], buf, sem)

    def drain(buf, sem):
        pltpu.make_async_copy(msg_hbm.at[pl.ds(0, SS)], buf, sem).wait()

    def scat(ch, buf):
        pltpu.sync_copy(buf, acc_sh.at[idx_v.at[ch]], add=True)
        pltpu.sync_copy(ones_v, cacc_sh.at[idx_v.at[ch]], add=True)

    fire(0, buf0, sem0)

    def pair(i, carry):
        fire(2 * i + 1, buf1, sem1)
        drain(buf0, sem0)
        scat(2 * i, buf0)

        @pl.when(i < NSP - 1)
        def _():
            fire(2 * i + 2, buf0, sem0)

        drain(buf1, sem1)
        scat(2 * i + 1, buf1)
        return carry

    lax.fori_loop(0, NSP, pair, 0)
    plsc.subcore_barrier()
    pltpu.sync_copy(acc_sh.at[pl.ds(s * RPT, RPT)], sums_hbm.at[c, pl.ds(s * RPT, RPT)])
    pltpu.sync_copy(cacc_sh.at[pl.ds(s * RPT, RPT)], cnt_hbm.at[c, pl.ds(s * RPT, RPT)])


BE = 2560


def _edge_mlp_body(g_ref, ea_ref, w1a_ref, b1a_ref, w1b_ref, b1b_ref, out_ref):
    g = g_ref[...].astype(jnp.float32)
    ea = ea_ref[...]
    h = jnp.dot(g, w1a_ref[0:D_IN, :], preferred_element_type=jnp.float32)
    h += jnp.dot(ea, w1a_ref[D_IN:D_IN + D_EDGE, :], preferred_element_type=jnp.float32)
    h = jax.nn.relu(h + b1a_ref[...])
    h = jnp.dot(h, w1b_ref[...], preferred_element_type=jnp.float32) + b1b_ref[...]
    out_ref[...] = jax.nn.relu(h)


def _edge_mlp(gathered, ea, W1a, b1a, W1b, b1b):
    grid = (EPAD // BE,)
    return pl.pallas_call(
        _edge_mlp_body,
        grid=grid,
        in_specs=[
            pl.BlockSpec((BE, D_IN), lambda i: (i, 0)),
            pl.BlockSpec((BE, D_EDGE), lambda i: (i, 0)),
            pl.BlockSpec((D_IN + D_EDGE, H), lambda i: (0, 0)),
            pl.BlockSpec((1, H), lambda i: (0, 0)),
            pl.BlockSpec((H, H), lambda i: (0, 0)),
            pl.BlockSpec((1, H), lambda i: (0, 0)),
        ],
        out_specs=pl.BlockSpec((BE, H), lambda i: (i, 0)),
        out_shape=jax.ShapeDtypeStruct((EPAD, H), jnp.float32),
        compiler_params=pltpu.CompilerParams(
            dimension_semantics=("arbitrary",)),
    )(gathered, ea, W1a, b1a, W1b, b1b)


BN = 2000


def _node_mlp_body(x_ref, sums_ref, cnt_ref, batch_ref, u_ref,
                   w2a_ref, b2a_ref, w2b_ref, b2b_ref, out_ref):
    x = x_ref[...]
    sums = sums_ref[0] + sums_ref[1]
    cnt = cnt_ref[0] + cnt_ref[1]
    mean = sums / jnp.maximum(cnt, 1.0)
    b = batch_ref[...]
    iota_g = lax.broadcasted_iota(jnp.int32, (1, G), 1)
    onehot = (b == iota_g).astype(jnp.float32)
    ug = jnp.dot(onehot, u_ref[...], preferred_element_type=jnp.float32)
    h = jnp.dot(x, w2a_ref[0:D_IN, :], preferred_element_type=jnp.float32)
    h += jnp.dot(mean, w2a_ref[D_IN:D_IN + H, :], preferred_element_type=jnp.float32)
    h += jnp.dot(ug, w2a_ref[D_IN + H:D_IN + H + U_DIM, :],
                 preferred_element_type=jnp.float32)
    h = jax.nn.relu(h + b2a_ref[...])
    out_ref[...] = jnp.dot(h, w2b_ref[...], preferred_element_type=jnp.float32) \
        + b2b_ref[...]


def _node_mlp(x, sums, cnt, batch2d, u, W2a, b2a, W2b, b2b):
    grid = (N // BN,)
    return pl.pallas_call(
        _node_mlp_body,
        grid=grid,
        in_specs=[
            pl.BlockSpec((BN, D_IN), lambda i: (i, 0)),
            pl.BlockSpec((NC, BN, H), lambda i: (0, i, 0)),
            pl.BlockSpec((NC, BN, 1), lambda i: (0, i, 0)),
            pl.BlockSpec((BN, 1), lambda i: (i, 0)),
            pl.BlockSpec((G, U_DIM), lambda i: (0, 0)),
            pl.BlockSpec((D_IN + H + U_DIM, H), lambda i: (0, 0)),
            pl.BlockSpec((1, H), lambda i: (0, 0)),
            pl.BlockSpec((H, D_OUT), lambda i: (0, 0)),
            pl.BlockSpec((1, D_OUT), lambda i: (0, 0)),
        ],
        out_specs=pl.BlockSpec((BN, D_OUT), lambda i: (i, 0)),
        out_shape=jax.ShapeDtypeStruct((N, D_OUT), jnp.float32),
        compiler_params=pltpu.CompilerParams(
            dimension_semantics=("arbitrary",)),
    )(x, sums, cnt, batch2d, u, W2a, b2a, W2b, b2b)


def kernel(x, edge_index, edge_attr, u, batch, W1a, b1a, W1b, b1b,
           W2a, b2a, W2b, b2b):
    row = edge_index[0]
    col = edge_index[1]
    pad = EPAD - E
    col_p = jnp.concatenate([col, jnp.zeros((pad,), jnp.int32)]).reshape(EPAD // 128, 128)
    row_p = jnp.concatenate([row, jnp.full((pad,), TRASH, jnp.int32)]).reshape(EPAD // 128, 128)
    ea_p = jnp.concatenate([edge_attr, jnp.zeros((pad, D_EDGE), jnp.float32)], axis=0)

    gathered = _sc_gather(x, col_p)
    msg = _edge_mlp(gathered, ea_p, W1a, b1a.reshape(1, H), W1b, b1b.reshape(1, H))

    zrows = jnp.zeros((NACC, H), jnp.float32)
    zcnt = jnp.zeros((NACC,), jnp.float32)
    ones = jnp.ones((128,), jnp.float32)
    sums, cnt = _sc_scatter(msg, row_p, zrows, zcnt, ones)

    out = _node_mlp(x, sums[:, :N, :], cnt[:, :N].reshape(NC, N, 1), batch.reshape(N, 1), u,
                    W2a, b2a.reshape(1, H), W2b, b2b.reshape(1, D_OUT))
    return out


# speedup vs baseline: 2.0266x; 1.0116x over previous
import functools

import jax
import jax.numpy as jnp
from jax import lax
from jax.experimental import pallas as pl
from jax.experimental.pallas import tpu as pltpu
from jax.experimental.pallas import tpu_sc as plsc

N = 10000
E = 320000
D_IN = 128
D_EDGE = 16
H = 128
D_OUT = 128
U_DIM = 64
G = 16

NC, NS = 2, 16
NW = NC * NS
EPW = 10240
EPAD = NW * EPW
GPW = EPW // 128
GS = 256
NGC = EPW // GS
NGP = NGC // 2
SS = 128
NSC = EPW // SS
NSP = NSC // 2
NACC = 10240
TRASH = N
RPT = NACC // NS

_sc_mesh = plsc.VectorSubcoreMesh(core_axis_name="c", subcore_axis_name="s",
                                  num_cores=NC, num_subcores=NS)


@functools.partial(
    pl.kernel, mesh=_sc_mesh,
    out_type=jax.ShapeDtypeStruct((EPAD, D_IN), jnp.float32),
    scratch_types=[
        pltpu.VMEM((GPW, 128), jnp.int32),
        pltpu.VMEM((128, D_IN), jnp.float32),
        pltpu.VMEM((128, D_IN), jnp.float32),
        pltpu.SemaphoreType.DMA,
        pltpu.SemaphoreType.DMA,
        pltpu.VMEM_SHARED((NACC, D_IN), jnp.float32),
    ],
)
def _sc_gather(x_hbm, col_hbm, out_hbm, idx_v, buf0, buf1, sem0, sem1, xs_sh):
    s = lax.axis_index("s")
    wid = s * NC + lax.axis_index("c")

    @pl.when(s < NS - 1)
    def _():
        pltpu.sync_copy(x_hbm.at[pl.ds(s * RPT, RPT)], xs_sh.at[pl.ds(s * RPT, RPT)])

    @pl.when(s == NS - 1)
    def _():
        pltpu.sync_copy(x_hbm.at[pl.ds((NS - 1) * RPT, N - (NS - 1) * RPT)],
                        xs_sh.at[pl.ds((NS - 1) * RPT, N - (NS - 1) * RPT)])

    pltpu.sync_copy(col_hbm.at[pl.ds(wid * GPW, GPW)], idx_v)
    plsc.subcore_barrier()

    def fire(c, buf, sem):
        pltpu.async_copy(xs_sh.at[idx_v.at[c]], buf, sem)

    def drain(buf, sem):
        pltpu.make_async_copy(xs_sh.at[idx_v.at[0]], buf, sem).wait()

    def store(c, buf):
        pltpu.sync_copy(buf, out_hbm.at[pl.ds(wid * EPW + c * 128, 128)])

    fire(0, buf0, sem0)

    def pair(i, carry):
        fire(2 * i + 1, buf1, sem1)
        drain(buf0, sem0)
        store(2 * i, buf0)

        @pl.when(i < GPW // 2 - 1)
        def _():
            fire(2 * i + 2, buf0, sem0)

        drain(buf1, sem1)
        store(2 * i + 1, buf1)
        return carry

    lax.fori_loop(0, GPW // 2, pair, 0)


@functools.partial(
    pl.kernel, mesh=_sc_mesh,
    out_type=(
        jax.ShapeDtypeStruct((NC, NACC, H), jnp.float32),
        jax.ShapeDtypeStruct((NC, NACC), jnp.float32),
    ),
    scratch_types=[
        pltpu.VMEM((GPW, 128), jnp.int32),
        pltpu.VMEM((SS, H), jnp.float32),
        pltpu.VMEM((SS, H), jnp.float32),
        pltpu.VMEM((128,), jnp.float32),
        pltpu.SemaphoreType.DMA,
        pltpu.SemaphoreType.DMA,
        pltpu.VMEM_SHARED((NACC, H), jnp.float32),
        pltpu.VMEM_SHARED((NACC,), jnp.float32),
    ],
)
def _sc_scatter(msg_hbm, row_hbm, zrows_hbm, zcnt_hbm, ones_hbm,
                sums_hbm, cnt_hbm, idx_v, buf0, buf1, ones_v, sem0, sem1,
                acc_sh, cacc_sh):
    c = lax.axis_index("c")
    s = lax.axis_index("s")
    wid = s * NC + c
    pltpu.sync_copy(zrows_hbm.at[pl.ds(s * RPT, RPT)], acc_sh.at[pl.ds(s * RPT, RPT)])
    pltpu.sync_copy(zcnt_hbm.at[pl.ds(s * RPT, RPT)], cacc_sh.at[pl.ds(s * RPT, RPT)])
    pltpu.sync_copy(ones_hbm, ones_v)
    pltpu.sync_copy(row_hbm.at[pl.ds(wid * GPW, GPW)], idx_v)
    plsc.subcore_barrier()

    def fire(ch, buf, sem):
        pltpu.async_copy(msg_hbm.at[pl.ds(wid * EPW + ch * SS, SS)], buf, sem)

    def drain(buf, sem):
        pltpu.make_async_copy(msg_hbm.at[pl.ds(0, SS)], buf, sem).wait()

    def scat(ch, buf):
        pltpu.sync_copy(buf, acc_sh.at[idx_v.at[ch]], add=True)
        pltpu.sync_copy(ones_v, cacc_sh.at[idx_v.at[ch]], add=True)

    fire(0, buf0, sem0)

    def pair(i, carry):
        fire(2 * i + 1, buf1, sem1)
        drain(buf0, sem0)
        scat(2 * i, buf0)

        @pl.when(i < NSP - 1)
        def _():
            fire(2 * i + 2, buf0, sem0)

        drain(buf1, sem1)
        scat(2 * i + 1, buf1)
        return carry

    lax.fori_loop(0, NSP, pair, 0)
    plsc.subcore_barrier()
    pltpu.sync_copy(acc_sh.at[pl.ds(s * RPT, RPT)], sums_hbm.at[c, pl.ds(s * RPT, RPT)])
    pltpu.sync_copy(cacc_sh.at[pl.ds(s * RPT, RPT)], cnt_hbm.at[c, pl.ds(s * RPT, RPT)])


BE = 2560


def _edge_mlp_body(g_ref, ea_ref, w1a_ref, b1a_ref, w1b_ref, b1b_ref, out_ref):
    g = g_ref[...].astype(jnp.float32)
    ea = ea_ref[...]
    h = jnp.dot(g, w1a_ref[0:D_IN, :], preferred_element_type=jnp.float32)
    h += jnp.dot(ea, w1a_ref[D_IN:D_IN + D_EDGE, :], preferred_element_type=jnp.float32)
    h = jax.nn.relu(h + b1a_ref[...])
    h = jnp.dot(h, w1b_ref[...], preferred_element_type=jnp.float32) + b1b_ref[...]
    out_ref[...] = jax.nn.relu(h)


def _edge_mlp(gathered, ea, W1a, b1a, W1b, b1b):
    grid = (EPAD // BE,)
    return pl.pallas_call(
        _edge_mlp_body,
        grid=grid,
        in_specs=[
            pl.BlockSpec((BE, D_IN), lambda i: (i, 0)),
            pl.BlockSpec((BE, D_EDGE), lambda i: (i, 0)),
            pl.BlockSpec((D_IN + D_EDGE, H), lambda i: (0, 0)),
            pl.BlockSpec((1, H), lambda i: (0, 0)),
            pl.BlockSpec((H, H), lambda i: (0, 0)),
            pl.BlockSpec((1, H), lambda i: (0, 0)),
        ],
        out_specs=pl.BlockSpec((BE, H), lambda i: (i, 0)),
        out_shape=jax.ShapeDtypeStruct((EPAD, H), jnp.float32),
        compiler_params=pltpu.CompilerParams(
            dimension_semantics=("arbitrary",)),
    )(gathered, ea, W1a, b1a, W1b, b1b)


BN = 2000


def _node_mlp_body(x_ref, sums_ref, cnt_ref, batch_ref, u_ref,
                   w2a_ref, b2a_ref, w2b_ref, b2b_ref, out_ref):
    x = x_ref[...]
    sums = sums_ref[0] + sums_ref[1]
    cnt = cnt_ref[0] + cnt_ref[1]
    mean = sums / jnp.maximum(cnt, 1.0)
    b = batch_ref[...]
    iota_g = lax.broadcasted_iota(jnp.int32, (1, G), 1)
    onehot = (b == iota_g).astype(jnp.float32)
    ug = jnp.dot(onehot, u_ref[...], preferred_element_type=jnp.float32)
    h = jnp.dot(x, w2a_ref[0:D_IN, :], preferred_element_type=jnp.float32)
    h += jnp.dot(mean, w2a_ref[D_IN:D_IN + H, :], preferred_element_type=jnp.float32)
    h += jnp.dot(ug, w2a_ref[D_IN + H:D_IN + H + U_DIM, :],
                 preferred_element_type=jnp.float32)
    h = jax.nn.relu(h + b2a_ref[...])
    out_ref[...] = jnp.dot(h, w2b_ref[...], preferred_element_type=jnp.float32) \
        + b2b_ref[...]


def _node_mlp(x, sums, cnt, batch2d, u, W2a, b2a, W2b, b2b):
    grid = (N // BN,)
    return pl.pallas_call(
        _node_mlp_body,
        grid=grid,
        in_specs=[
            pl.BlockSpec((BN, D_IN), lambda i: (i, 0)),
            pl.BlockSpec((NC, BN, H), lambda i: (0, i, 0)),
            pl.BlockSpec((NC, BN, 1), lambda i: (0, i, 0)),
            pl.BlockSpec((BN, 1), lambda i: (i, 0)),
            pl.BlockSpec((G, U_DIM), lambda i: (0, 0)),
            pl.BlockSpec((D_IN + H + U_DIM, H), lambda i: (0, 0)),
            pl.BlockSpec((1, H), lambda i: (0, 0)),
            pl.BlockSpec((H, D_OUT), lambda i: (0, 0)),
            pl.BlockSpec((1, D_OUT), lambda i: (0, 0)),
        ],
        out_specs=pl.BlockSpec((BN, D_OUT), lambda i: (i, 0)),
        out_shape=jax.ShapeDtypeStruct((N, D_OUT), jnp.float32),
        compiler_params=pltpu.CompilerParams(
            dimension_semantics=("arbitrary",)),
    )(x, sums, cnt, batch2d, u, W2a, b2a, W2b, b2b)


def kernel(x, edge_index, edge_attr, u, batch, W1a, b1a, W1b, b1b,
           W2a, b2a, W2b, b2b):
    row = edge_index[0]
    col = edge_index[1]
    pad = EPAD - E
    col_p = jnp.concatenate([col, jnp.zeros((pad,), jnp.int32)]).reshape(EPAD // 128, 128)
    row_p = jnp.concatenate([row, jnp.full((pad,), TRASH, jnp.int32)]).reshape(EPAD // 128, 128)
    ea_p = jnp.concatenate([edge_attr, jnp.zeros((pad, D_EDGE), jnp.float32)], axis=0)

    gathered = _sc_gather(x, col_p)
    msg = _edge_mlp(gathered, ea_p, W1a, b1a.reshape(1, H), W1b, b1b.reshape(1, H))

    zrows = jnp.zeros((NACC, H), jnp.float32)
    zcnt = jnp.zeros((NACC,), jnp.float32)
    ones = jnp.ones((128,), jnp.float32)
    sums, cnt = _sc_scatter(msg, row_p, zrows, zcnt, ones)

    out = _node_mlp(x, sums, cnt.reshape(NC, NACC, 1), batch.reshape(N, 1), u,
                    W2a, b2a.reshape(1, H), W2b, b2b.reshape(1, D_OUT))
    return out


# speedup vs baseline: 2.0369x; 1.0051x over previous
import functools

import jax
import jax.numpy as jnp
from jax import lax
from jax.experimental import pallas as pl
from jax.experimental.pallas import tpu as pltpu
from jax.experimental.pallas import tpu_sc as plsc

N = 10000
E = 320000
D_IN = 128
D_EDGE = 16
H = 128
D_OUT = 128
U_DIM = 64
G = 16

NC, NS = 2, 16
NW = NC * NS
EPW = 10240
EPAD = NW * EPW
GPW = EPW // 128
NCHUNK = 2
EPAD2 = EPAD // NCHUNK
EPW2 = EPW // NCHUNK
GPW2 = GPW // NCHUNK
SS = 128
NSC = EPW // SS
NSP = NSC // 2
NACC = 10240
TRASH = N
RPT = NACC // NS

_sc_mesh = plsc.VectorSubcoreMesh(core_axis_name="c", subcore_axis_name="s",
                                  num_cores=NC, num_subcores=NS)


@functools.partial(
    pl.kernel, mesh=_sc_mesh,
    out_type=jax.ShapeDtypeStruct((EPAD2, D_IN), jnp.float32),
    scratch_types=[
        pltpu.VMEM((GPW2, 128), jnp.int32),
        pltpu.VMEM((128, D_IN), jnp.float32),
        pltpu.VMEM((128, D_IN), jnp.float32),
        pltpu.SemaphoreType.DMA,
        pltpu.SemaphoreType.DMA,
        pltpu.VMEM_SHARED((NACC, D_IN), jnp.float32),
    ],
)
def _sc_gather(x_hbm, col_hbm, out_hbm, idx_v, buf0, buf1, sem0, sem1, xs_sh):
    s = lax.axis_index("s")
    wid = s * NC + lax.axis_index("c")

    @pl.when(s < NS - 1)
    def _():
        pltpu.sync_copy(x_hbm.at[pl.ds(s * RPT, RPT)], xs_sh.at[pl.ds(s * RPT, RPT)])

    @pl.when(s == NS - 1)
    def _():
        pltpu.sync_copy(x_hbm.at[pl.ds((NS - 1) * RPT, N - (NS - 1) * RPT)],
                        xs_sh.at[pl.ds((NS - 1) * RPT, N - (NS - 1) * RPT)])

    pltpu.sync_copy(col_hbm.at[pl.ds(wid * GPW2, GPW2)], idx_v)
    plsc.subcore_barrier()

    def fire(c, buf, sem):
        pltpu.async_copy(xs_sh.at[idx_v.at[c]], buf, sem)

    def drain(buf, sem):
        pltpu.make_async_copy(xs_sh.at[idx_v.at[0]], buf, sem).wait()

    def store(c, buf):
        pltpu.sync_copy(buf, out_hbm.at[pl.ds(wid * EPW2 + c * 128, 128)])

    fire(0, buf0, sem0)

    def pair(i, carry):
        fire(2 * i + 1, buf1, sem1)
        drain(buf0, sem0)
        store(2 * i, buf0)

        @pl.when(i < GPW2 // 2 - 1)
        def _():
            fire(2 * i + 2, buf0, sem0)

        drain(buf1, sem1)
        store(2 * i + 1, buf1)
        return carry

    lax.fori_loop(0, GPW2 // 2, pair, 0)


@functools.partial(
    pl.kernel, mesh=_sc_mesh,
    out_type=(
        jax.ShapeDtypeStruct((NC, NACC, H), jnp.float32),
        jax.ShapeDtypeStruct((NC, NACC), jnp.float32),
    ),
    scratch_types=[
        pltpu.VMEM((GPW, 128), jnp.int32),
        pltpu.VMEM((SS, H), jnp.float32),
        pltpu.VMEM((SS, H), jnp.float32),
        pltpu.VMEM((128,), jnp.float32),
        pltpu.SemaphoreType.DMA,
        pltpu.SemaphoreType.DMA,
        pltpu.VMEM_SHARED((NACC, H), jnp.float32),
        pltpu.VMEM_SHARED((NACC,), jnp.float32),
    ],
)
def _sc_scatter(msg1_hbm, msg2_hbm, row_hbm, zrows_hbm, zcnt_hbm, ones_hbm,
                sums_hbm, cnt_hbm, idx_v, buf0, buf1, ones_v, sem0, sem1,
                acc_sh, cacc_sh):
    c = lax.axis_index("c")
    s = lax.axis_index("s")
    wid = s * NC + c
    pltpu.sync_copy(zrows_hbm.at[pl.ds(s * RPT, RPT)], acc_sh.at[pl.ds(s * RPT, RPT)])
    pltpu.sync_copy(zcnt_hbm.at[pl.ds(s * RPT, RPT)], cacc_sh.at[pl.ds(s * RPT, RPT)])
    pltpu.sync_copy(ones_hbm, ones_v)
    pltpu.sync_copy(row_hbm.at[pl.ds(wid * GPW, GPW)], idx_v)
    plsc.subcore_barrier()

    def run(msg_hbm, base):
        def fire(ch, buf, sem):
            pltpu.async_copy(msg_hbm.at[pl.ds(base + ch * SS, SS)], buf, sem)

        def drain(buf, sem):
            pltpu.make_async_copy(msg_hbm.at[pl.ds(0, SS)], buf, sem).wait()

        def scat(ch, buf):
            pltpu.sync_copy(buf, acc_sh.at[idx_v.at[ch]], add=True)
            pltpu.sync_copy(ones_v, cacc_sh.at[idx_v.at[ch]], add=True)

        fire(0, buf0, sem0)

        def pair(i, carry):
            fire(2 * i + 1, buf1, sem1)
            drain(buf0, sem0)
            scat(2 * i, buf0)

            @pl.when(i < NSP - 1)
            def _():
                fire(2 * i + 2, buf0, sem0)

            drain(buf1, sem1)
            scat(2 * i + 1, buf1)
            return carry

        lax.fori_loop(0, NSP, pair, 0)

    @pl.when(wid < NW // 2)
    def _():
        run(msg1_hbm, wid * EPW)

    @pl.when(wid >= NW // 2)
    def _():
        run(msg2_hbm, wid * EPW - EPAD2)

    plsc.subcore_barrier()
    pltpu.sync_copy(acc_sh.at[pl.ds(s * RPT, RPT)], sums_hbm.at[c, pl.ds(s * RPT, RPT)])
    pltpu.sync_copy(cacc_sh.at[pl.ds(s * RPT, RPT)], cnt_hbm.at[c, pl.ds(s * RPT, RPT)])


BE = 2560


def _edge_mlp_body(g_ref, ea_ref, w1a_ref, b1a_ref, w1b_ref, b1b_ref, out_ref):
    g = g_ref[...]
    ea = ea_ref[...]
    h = jnp.dot(g, w1a_ref[0:D_IN, :], preferred_element_type=jnp.float32)
    h += jnp.dot(ea, w1a_ref[D_IN:D_IN + D_EDGE, :], preferred_element_type=jnp.float32)
    h = jax.nn.relu(h + b1a_ref[...])
    h = jnp.dot(h, w1b_ref[...], preferred_element_type=jnp.float32) + b1b_ref[...]
    out_ref[...] = jax.nn.relu(h)


def _edge_mlp(gathered, ea, W1a, b1a, W1b, b1b):
    grid = (gathered.shape[0] // BE,)
    return pl.pallas_call(
        _edge_mlp_body,
        grid=grid,
        in_specs=[
            pl.BlockSpec((BE, D_IN), lambda i: (i, 0)),
            pl.BlockSpec((BE, D_EDGE), lambda i: (i, 0)),
            pl.BlockSpec((D_IN + D_EDGE, H), lambda i: (0, 0)),
            pl.BlockSpec((1, H), lambda i: (0, 0)),
            pl.BlockSpec((H, H), lambda i: (0, 0)),
            pl.BlockSpec((1, H), lambda i: (0, 0)),
        ],
        out_specs=pl.BlockSpec((BE, H), lambda i: (i, 0)),
        out_shape=jax.ShapeDtypeStruct((gathered.shape[0], H), jnp.float32),
        compiler_params=pltpu.CompilerParams(
            dimension_semantics=("arbitrary",)),
    )(gathered, ea, W1a, b1a, W1b, b1b)


BN = 2000


def _node_mlp_body(x_ref, sums_ref, cnt_ref, batch_ref, u_ref,
                   w2a_ref, b2a_ref, w2b_ref, b2b_ref, out_ref):
    x = x_ref[...]
    sums = sums_ref[0] + sums_ref[1]
    cnt = cnt_ref[0] + cnt_ref[1]
    mean = sums / jnp.maximum(cnt, 1.0)
    b = batch_ref[...]
    iota_g = lax.broadcasted_iota(jnp.int32, (1, G), 1)
    onehot = (b == iota_g).astype(jnp.float32)
    ug = jnp.dot(onehot, u_ref[...], preferred_element_type=jnp.float32)
    h = jnp.dot(x, w2a_ref[0:D_IN, :], preferred_element_type=jnp.float32)
    h += jnp.dot(mean, w2a_ref[D_IN:D_IN + H, :], preferred_element_type=jnp.float32)
    h += jnp.dot(ug, w2a_ref[D_IN + H:D_IN + H + U_DIM, :],
                 preferred_element_type=jnp.float32)
    h = jax.nn.relu(h + b2a_ref[...])
    out_ref[...] = jnp.dot(h, w2b_ref[...], preferred_element_type=jnp.float32) \
        + b2b_ref[...]


def _node_mlp(x, sums, cnt, batch2d, u, W2a, b2a, W2b, b2b):
    grid = (N // BN,)
    return pl.pallas_call(
        _node_mlp_body,
        grid=grid,
        in_specs=[
            pl.BlockSpec((BN, D_IN), lambda i: (i, 0)),
            pl.BlockSpec((NC, BN, H), lambda i: (0, i, 0)),
            pl.BlockSpec((NC, BN, 1), lambda i: (0, i, 0)),
            pl.BlockSpec((BN, 1), lambda i: (i, 0)),
            pl.BlockSpec((G, U_DIM), lambda i: (0, 0)),
            pl.BlockSpec((D_IN + H + U_DIM, H), lambda i: (0, 0)),
            pl.BlockSpec((1, H), lambda i: (0, 0)),
            pl.BlockSpec((H, D_OUT), lambda i: (0, 0)),
            pl.BlockSpec((1, D_OUT), lambda i: (0, 0)),
        ],
        out_specs=pl.BlockSpec((BN, D_OUT), lambda i: (i, 0)),
        out_shape=jax.ShapeDtypeStruct((N, D_OUT), jnp.float32),
        compiler_params=pltpu.CompilerParams(
            dimension_semantics=("arbitrary",)),
    )(x, sums, cnt, batch2d, u, W2a, b2a, W2b, b2b)


def kernel(x, edge_index, edge_attr, u, batch, W1a, b1a, W1b, b1b,
           W2a, b2a, W2b, b2b):
    row = edge_index[0]
    col = edge_index[1]
    pad = EPAD - E
    col_p = jnp.concatenate([col, jnp.zeros((pad,), jnp.int32)]).reshape(EPAD // 128, 128)
    row_p = jnp.concatenate([row, jnp.full((pad,), TRASH, jnp.int32)]).reshape(EPAD // 128, 128)
    ea_p = jnp.concatenate([edge_attr, jnp.zeros((pad, D_EDGE), jnp.float32)], axis=0)

    b1a2, b1b2 = b1a.reshape(1, H), b1b.reshape(1, H)
    g1 = _sc_gather(x, col_p[:EPAD2 // 128])
    g2 = _sc_gather(x, col_p[EPAD2 // 128:])
    msg1 = _edge_mlp(g1, ea_p[:EPAD2], W1a, b1a2, W1b, b1b2)
    msg2 = _edge_mlp(g2, ea_p[EPAD2:], W1a, b1a2, W1b, b1b2)

    zrows = jnp.zeros((NACC, H), jnp.float32)
    zcnt = jnp.zeros((NACC,), jnp.float32)
    ones = jnp.ones((128,), jnp.float32)
    sums, cnt = _sc_scatter(msg1, msg2, row_p, zrows, zcnt, ones)

    out = _node_mlp(x, sums, cnt.reshape(NC, NACC, 1), batch.reshape(N, 1), u,
                    W2a, b2a.reshape(1, H), W2b, b2b.reshape(1, D_OUT))
    return out


# speedup vs baseline: 2.2500x; 1.1047x over previous
import functools

import jax
import jax.numpy as jnp
from jax import lax
from jax.experimental import pallas as pl
from jax.experimental.pallas import tpu as pltpu
from jax.experimental.pallas import tpu_sc as plsc

N = 10000
E = 320000
D_IN = 128
D_EDGE = 16
H = 128
D_OUT = 128
U_DIM = 64
G = 16

NC, NS = 2, 16
NW = NC * NS
EPW = 10240
EPAD = NW * EPW
GPW = EPW // 128
NCHUNK = 2
EPAD2 = EPAD // NCHUNK
EPW2 = EPW // NCHUNK
GPW2 = GPW // NCHUNK
SS = 128
NSC = EPW // SS
NSP = NSC // 2
NACC = 10240
TRASH = N
RPT = NACC // NS

_sc_mesh = plsc.VectorSubcoreMesh(core_axis_name="c", subcore_axis_name="s",
                                  num_cores=NC, num_subcores=NS)


@functools.partial(
    pl.kernel, mesh=_sc_mesh,
    out_type=jax.ShapeDtypeStruct((EPAD2, D_IN), jnp.float32),
    scratch_types=[
        pltpu.VMEM((GPW2, 128), jnp.int32),
        pltpu.VMEM((128, D_IN), jnp.float32),
        pltpu.VMEM((128, D_IN), jnp.float32),
        pltpu.SemaphoreType.DMA,
        pltpu.SemaphoreType.DMA,
        pltpu.VMEM_SHARED((NACC, D_IN), jnp.float32),
    ],
)
def _sc_gather(x_hbm, col_hbm, out_hbm, idx_v, buf0, buf1, sem0, sem1, xs_sh):
    s = lax.axis_index("s")
    wid = s * NC + lax.axis_index("c")

    @pl.when(s < NS - 1)
    def _():
        pltpu.sync_copy(x_hbm.at[pl.ds(s * RPT, RPT)], xs_sh.at[pl.ds(s * RPT, RPT)])

    @pl.when(s == NS - 1)
    def _():
        pltpu.sync_copy(x_hbm.at[pl.ds((NS - 1) * RPT, N - (NS - 1) * RPT)],
                        xs_sh.at[pl.ds((NS - 1) * RPT, N - (NS - 1) * RPT)])

    pltpu.sync_copy(col_hbm.at[pl.ds(wid * GPW2, GPW2)], idx_v)
    plsc.subcore_barrier()

    def fire(c, buf, sem):
        pltpu.async_copy(xs_sh.at[idx_v.at[c]], buf, sem)

    def drain(buf, sem):
        pltpu.make_async_copy(xs_sh.at[idx_v.at[0]], buf, sem).wait()

    def store(c, buf):
        pltpu.sync_copy(buf, out_hbm.at[pl.ds(wid * EPW2 + c * 128, 128)])

    fire(0, buf0, sem0)

    def pair(i, carry):
        fire(2 * i + 1, buf1, sem1)
        drain(buf0, sem0)
        store(2 * i, buf0)

        @pl.when(i < GPW2 // 2 - 1)
        def _():
            fire(2 * i + 2, buf0, sem0)

        drain(buf1, sem1)
        store(2 * i + 1, buf1)
        return carry

    lax.fori_loop(0, GPW2 // 2, pair, 0)


@functools.partial(
    pl.kernel, mesh=_sc_mesh,
    out_type=(
        jax.ShapeDtypeStruct((NC, NACC, H), jnp.float32),
        jax.ShapeDtypeStruct((NC, NACC), jnp.float32),
    ),
    scratch_types=[
        pltpu.VMEM((GPW, 128), jnp.int32),
        pltpu.VMEM((SS, H), jnp.float32),
        pltpu.VMEM((SS, H), jnp.float32),
        pltpu.VMEM((128,), jnp.float32),
        pltpu.SemaphoreType.DMA,
        pltpu.SemaphoreType.DMA,
        pltpu.VMEM_SHARED((NACC, H), jnp.float32),
        pltpu.VMEM_SHARED((NACC,), jnp.float32),
    ],
)
def _sc_scatter(msg1_hbm, msg2_hbm, row_hbm, zrows_hbm, zcnt_hbm, ones_hbm,
                sums_hbm, cnt_hbm, idx_v, buf0, buf1, ones_v, sem0, sem1,
                acc_sh, cacc_sh):
    c = lax.axis_index("c")
    s = lax.axis_index("s")
    wid = s * NC + c
    pltpu.sync_copy(zrows_hbm.at[pl.ds(s * RPT, RPT)], acc_sh.at[pl.ds(s * RPT, RPT)])
    pltpu.sync_copy(zcnt_hbm.at[pl.ds(s * RPT, RPT)], cacc_sh.at[pl.ds(s * RPT, RPT)])
    pltpu.sync_copy(ones_hbm, ones_v)
    pltpu.sync_copy(row_hbm.at[pl.ds(wid * GPW, GPW)], idx_v)
    plsc.subcore_barrier()

    def run(msg_hbm, base):
        def fire(ch, buf, sem):
            pltpu.async_copy(msg_hbm.at[pl.ds(base + ch * SS, SS)], buf, sem)

        def drain(buf, sem):
            pltpu.make_async_copy(msg_hbm.at[pl.ds(0, SS)], buf, sem).wait()

        def scat(ch, buf):
            pltpu.sync_copy(buf, acc_sh.at[idx_v.at[ch]], add=True)
            pltpu.sync_copy(ones_v, cacc_sh.at[idx_v.at[ch]], add=True)

        fire(0, buf0, sem0)

        def pair(i, carry):
            fire(2 * i + 1, buf1, sem1)
            drain(buf0, sem0)
            scat(2 * i, buf0)

            @pl.when(i < NSP - 1)
            def _():
                fire(2 * i + 2, buf0, sem0)

            drain(buf1, sem1)
            scat(2 * i + 1, buf1)
            return carry

        lax.fori_loop(0, NSP, pair, 0)

    @pl.when(wid < NW // 2)
    def _():
        run(msg1_hbm, wid * EPW)

    @pl.when(wid >= NW // 2)
    def _():
        run(msg2_hbm, wid * EPW - EPAD2)

    plsc.subcore_barrier()
    pltpu.sync_copy(acc_sh.at[pl.ds(s * RPT, RPT)], sums_hbm.at[c, pl.ds(s * RPT, RPT)])
    pltpu.sync_copy(cacc_sh.at[pl.ds(s * RPT, RPT)], cnt_hbm.at[c, pl.ds(s * RPT, RPT)])


BE = 2560


def _edge_mlp_body(g_ref, ea_ref, w1a_ref, b1a_ref, w1b_ref, b1b_ref, out_ref):
    g = g_ref[...]
    ea = ea_ref[...]
    h = jnp.dot(g, w1a_ref[0:D_IN, :], preferred_element_type=jnp.float32)
    h += jnp.dot(ea, w1a_ref[D_IN:D_IN + D_EDGE, :], preferred_element_type=jnp.float32)
    h = jax.nn.relu(h + b1a_ref[...])
    h = jnp.dot(h, w1b_ref[...], preferred_element_type=jnp.float32) + b1b_ref[...]
    out_ref[...] = jax.nn.relu(h)


def _edge_mlp(gathered, ea, off, nblk, W1a, b1a, W1b, b1b):
    grid = (nblk,)
    return pl.pallas_call(
        _edge_mlp_body,
        grid=grid,
        in_specs=[
            pl.BlockSpec((BE, D_IN), lambda i: (i, 0)),
            pl.BlockSpec((BE, D_EDGE), lambda i: (i + off, 0)),
            pl.BlockSpec((D_IN + D_EDGE, H), lambda i: (0, 0)),
            pl.BlockSpec((1, H), lambda i: (0, 0)),
            pl.BlockSpec((H, H), lambda i: (0, 0)),
            pl.BlockSpec((1, H), lambda i: (0, 0)),
        ],
        out_specs=pl.BlockSpec((BE, H), lambda i: (i, 0)),
        out_shape=jax.ShapeDtypeStruct((gathered.shape[0], H), jnp.float32),
        compiler_params=pltpu.CompilerParams(
            dimension_semantics=("arbitrary",)),
    )(gathered, ea, W1a, b1a, W1b, b1b)


BN = 2000


def _node_mlp_body(x_ref, sums_ref, cnt_ref, batch_ref, u_ref,
                   w2a_ref, b2a_ref, w2b_ref, b2b_ref, out_ref):
    x = x_ref[...]
    sums = sums_ref[0] + sums_ref[1]
    cnt = cnt_ref[0] + cnt_ref[1]
    mean = sums / jnp.maximum(cnt, 1.0)
    b = batch_ref[...]
    iota_g = lax.broadcasted_iota(jnp.int32, (1, G), 1)
    onehot = (b == iota_g).astype(jnp.float32)
    ug = jnp.dot(onehot, u_ref[...], preferred_element_type=jnp.float32)
    h = jnp.dot(x, w2a_ref[0:D_IN, :], preferred_element_type=jnp.float32)
    h += jnp.dot(mean, w2a_ref[D_IN:D_IN + H, :], preferred_element_type=jnp.float32)
    h += jnp.dot(ug, w2a_ref[D_IN + H:D_IN + H + U_DIM, :],
                 preferred_element_type=jnp.float32)
    h = jax.nn.relu(h + b2a_ref[...])
    out_ref[...] = jnp.dot(h, w2b_ref[...], preferred_element_type=jnp.float32) \
        + b2b_ref[...]


def _node_mlp(x, sums, cnt, batch2d, u, W2a, b2a, W2b, b2b):
    grid = (N // BN,)
    return pl.pallas_call(
        _node_mlp_body,
        grid=grid,
        in_specs=[
            pl.BlockSpec((BN, D_IN), lambda i: (i, 0)),
            pl.BlockSpec((NC, BN, H), lambda i: (0, i, 0)),
            pl.BlockSpec((NC, BN, 1), lambda i: (0, i, 0)),
            pl.BlockSpec((BN, 1), lambda i: (i, 0)),
            pl.BlockSpec((G, U_DIM), lambda i: (0, 0)),
            pl.BlockSpec((D_IN + H + U_DIM, H), lambda i: (0, 0)),
            pl.BlockSpec((1, H), lambda i: (0, 0)),
            pl.BlockSpec((H, D_OUT), lambda i: (0, 0)),
            pl.BlockSpec((1, D_OUT), lambda i: (0, 0)),
        ],
        out_specs=pl.BlockSpec((BN, D_OUT), lambda i: (i, 0)),
        out_shape=jax.ShapeDtypeStruct((N, D_OUT), jnp.float32),
        compiler_params=pltpu.CompilerParams(
            dimension_semantics=("arbitrary",)),
    )(x, sums, cnt, batch2d, u, W2a, b2a, W2b, b2b)


def kernel(x, edge_index, edge_attr, u, batch, W1a, b1a, W1b, b1b,
           W2a, b2a, W2b, b2b):
    row = edge_index[0]
    col = edge_index[1]
    pad = EPAD - E
    col_p = jnp.concatenate([col, jnp.zeros((pad,), jnp.int32)]).reshape(EPAD // 128, 128)
    row_p = jnp.concatenate([row, jnp.full((pad,), TRASH, jnp.int32)]).reshape(EPAD // 128, 128)

    b1a2, b1b2 = b1a.reshape(1, H), b1b.reshape(1, H)
    g1 = _sc_gather(x, col_p[:EPAD2 // 128])
    g2 = _sc_gather(x, col_p[EPAD2 // 128:])
    msg1 = _edge_mlp(g1, edge_attr, 0, EPAD2 // BE, W1a, b1a2, W1b, b1b2)
    msg2 = _edge_mlp(g2, edge_attr, EPAD2 // BE, (E - EPAD2) // BE,
                     W1a, b1a2, W1b, b1b2)

    zrows = jnp.zeros((NACC, H), jnp.float32)
    zcnt = jnp.zeros((NACC,), jnp.float32)
    ones = jnp.ones((128,), jnp.float32)
    sums, cnt = _sc_scatter(msg1, msg2, row_p, zrows, zcnt, ones)

    out = _node_mlp(x, sums, cnt.reshape(NC, NACC, 1), batch.reshape(N, 1), u,
                    W2a, b2a.reshape(1, H), W2b, b2b.reshape(1, D_OUT))
    return out


# speedup vs baseline: 2.6894x; 1.1953x over previous
import functools

import jax
import jax.numpy as jnp
from jax import lax
from jax.experimental import pallas as pl
from jax.experimental.pallas import tpu as pltpu
from jax.experimental.pallas import tpu_sc as plsc

N = 10000
E = 320000
D_IN = 128
D_EDGE = 16
H = 128
D_OUT = 128
U_DIM = 64
G = 16

NC, NS = 2, 16
NW = NC * NS
EPW = 10240
EPAD = NW * EPW
GPW = EPW // 128
NCHUNK = 2
EPAD2 = EPAD // NCHUNK
EPW2 = EPW // NCHUNK
GPW2 = GPW // NCHUNK
SS = 128
NSC = EPW // SS
NSP = NSC // 2
NACC = 10240
TRASH = N
RPT = NACC // NS

_sc_mesh = plsc.VectorSubcoreMesh(core_axis_name="c", subcore_axis_name="s",
                                  num_cores=NC, num_subcores=NS)


@functools.partial(
    pl.kernel, mesh=_sc_mesh,
    out_type=jax.ShapeDtypeStruct((EPAD2, D_IN), jnp.float32),
    scratch_types=[
        pltpu.VMEM((GPW2, 128), jnp.int32),
        pltpu.VMEM((128, D_IN), jnp.float32),
        pltpu.VMEM((128, D_IN), jnp.float32),
        pltpu.SemaphoreType.DMA,
        pltpu.SemaphoreType.DMA,
        pltpu.VMEM_SHARED((NACC, D_IN), jnp.float32),
    ],
)
def _sc_gather(x_hbm, col_hbm, out_hbm, idx_v, buf0, buf1, sem0, sem1, xs_sh):
    s = lax.axis_index("s")
    wid = s * NC + lax.axis_index("c")

    @pl.when(s < NS - 1)
    def _():
        pltpu.sync_copy(x_hbm.at[pl.ds(s * RPT, RPT)], xs_sh.at[pl.ds(s * RPT, RPT)])

    @pl.when(s == NS - 1)
    def _():
        pltpu.sync_copy(x_hbm.at[pl.ds((NS - 1) * RPT, N - (NS - 1) * RPT)],
                        xs_sh.at[pl.ds((NS - 1) * RPT, N - (NS - 1) * RPT)])

    pltpu.sync_copy(col_hbm.at[pl.ds(wid * GPW2, GPW2)], idx_v)
    plsc.subcore_barrier()

    def fire(c, buf, sem):
        pltpu.async_copy(xs_sh.at[idx_v.at[c]], buf, sem)

    def drain(buf, sem):
        pltpu.make_async_copy(xs_sh.at[idx_v.at[0]], buf, sem).wait()

    def store(c, buf):
        pltpu.sync_copy(buf, out_hbm.at[pl.ds(wid * EPW2 + c * 128, 128)])

    fire(0, buf0, sem0)

    def pair(i, carry):
        fire(2 * i + 1, buf1, sem1)
        drain(buf0, sem0)
        store(2 * i, buf0)

        @pl.when(i < GPW2 // 2 - 1)
        def _():
            fire(2 * i + 2, buf0, sem0)

        drain(buf1, sem1)
        store(2 * i + 1, buf1)
        return carry

    lax.fori_loop(0, GPW2 // 2, pair, 0)


@functools.partial(
    pl.kernel, mesh=_sc_mesh,
    out_type=(
        jax.ShapeDtypeStruct((NC, NACC, H), jnp.float32),
        jax.ShapeDtypeStruct((NC, NACC), jnp.float32),
    ),
    scratch_types=[
        pltpu.VMEM((GPW, 128), jnp.int32),
        pltpu.VMEM((SS, H), jnp.float32),
        pltpu.VMEM((SS, H), jnp.float32),
        pltpu.VMEM((128,), jnp.float32),
        pltpu.SemaphoreType.DMA,
        pltpu.SemaphoreType.DMA,
        pltpu.VMEM_SHARED((NACC, H), jnp.float32),
        pltpu.VMEM_SHARED((NACC,), jnp.float32),
    ],
)
def _sc_scatter(msg1_hbm, msg2_hbm, row_hbm, zrows_hbm, zcnt_hbm, ones_hbm,
                sums_hbm, cnt_hbm, idx_v, buf0, buf1, ones_v, sem0, sem1,
                acc_sh, cacc_sh):
    c = lax.axis_index("c")
    s = lax.axis_index("s")
    wid = s * NC + c
    pltpu.sync_copy(zrows_hbm.at[pl.ds(s * RPT, RPT)], acc_sh.at[pl.ds(s * RPT, RPT)])
    pltpu.sync_copy(zcnt_hbm.at[pl.ds(s * RPT, RPT)], cacc_sh.at[pl.ds(s * RPT, RPT)])
    pltpu.sync_copy(ones_hbm, ones_v)
    pltpu.sync_copy(row_hbm.at[pl.ds(wid * GPW, GPW)], idx_v)
    plsc.subcore_barrier()

    def run(msg_hbm, base):
        def fire(ch, buf, sem):
            pltpu.async_copy(msg_hbm.at[pl.ds(base + ch * SS, SS)], buf, sem)

        def drain(buf, sem):
            pltpu.make_async_copy(msg_hbm.at[pl.ds(0, SS)], buf, sem).wait()

        def scat(ch, buf):
            pltpu.sync_copy(buf, acc_sh.at[idx_v.at[ch]], add=True)
            pltpu.sync_copy(ones_v, cacc_sh.at[idx_v.at[ch]], add=True)

        fire(0, buf0, sem0)

        def pair(i, carry):
            fire(2 * i + 1, buf1, sem1)
            drain(buf0, sem0)
            scat(2 * i, buf0)

            @pl.when(i < NSP - 1)
            def _():
                fire(2 * i + 2, buf0, sem0)

            drain(buf1, sem1)
            scat(2 * i + 1, buf1)
            return carry

        lax.fori_loop(0, NSP, pair, 0)

    @pl.when(wid < NW // 2)
    def _():
        run(msg1_hbm, wid * EPW)

    @pl.when(wid >= NW // 2)
    def _():
        run(msg2_hbm, wid * EPW - EPAD2)

    plsc.subcore_barrier()
    pltpu.sync_copy(acc_sh.at[pl.ds(s * RPT, RPT)], sums_hbm.at[c, pl.ds(s * RPT, RPT)])
    pltpu.sync_copy(cacc_sh.at[pl.ds(s * RPT, RPT)], cnt_hbm.at[c, pl.ds(s * RPT, RPT)])


BE = 2560


def _edge_mlp_body(g_ref, ea_ref, w1a_ref, b1a_ref, w1b_ref, b1b_ref, out_ref):
    g = g_ref[...]
    ea_t = ea_ref[...]
    h = jnp.dot(g, w1a_ref[0:D_IN, :], preferred_element_type=jnp.float32)
    h += jax.lax.dot_general(
        ea_t, w1a_ref[D_IN:D_IN + D_EDGE, :], (((0,), (0,)), ((), ())),
        preferred_element_type=jnp.float32)
    h = jax.nn.relu(h + b1a_ref[...])
    h = jnp.dot(h, w1b_ref[...], preferred_element_type=jnp.float32) + b1b_ref[...]
    out_ref[...] = jax.nn.relu(h)


def _edge_mlp(gathered, ea, off, nblk, W1a, b1a, W1b, b1b):
    grid = (nblk,)
    return pl.pallas_call(
        _edge_mlp_body,
        grid=grid,
        in_specs=[
            pl.BlockSpec((BE, D_IN), lambda i: (i, 0)),
            pl.BlockSpec((D_EDGE, BE), lambda i: (0, i + off)),
            pl.BlockSpec((D_IN + D_EDGE, H), lambda i: (0, 0)),
            pl.BlockSpec((1, H), lambda i: (0, 0)),
            pl.BlockSpec((H, H), lambda i: (0, 0)),
            pl.BlockSpec((1, H), lambda i: (0, 0)),
        ],
        out_specs=pl.BlockSpec((BE, H), lambda i: (i, 0)),
        out_shape=jax.ShapeDtypeStruct((gathered.shape[0], H), jnp.float32),
        compiler_params=pltpu.CompilerParams(
            dimension_semantics=("arbitrary",)),
    )(gathered, ea, W1a, b1a, W1b, b1b)


BN = 2000


def _node_mlp_body(x_ref, sums_ref, cnt_ref, batch_ref, u_ref,
                   w2a_ref, b2a_ref, w2b_ref, b2b_ref, out_ref):
    x = x_ref[...]
    sums = sums_ref[0] + sums_ref[1]
    cnt = cnt_ref[0] + cnt_ref[1]
    mean = sums / jnp.maximum(cnt, 1.0)
    b = batch_ref[...]
    iota_g = lax.broadcasted_iota(jnp.int32, (1, G), 1)
    onehot = (b == iota_g).astype(jnp.float32)
    ug = jnp.dot(onehot, u_ref[...], preferred_element_type=jnp.float32)
    h = jnp.dot(x, w2a_ref[0:D_IN, :], preferred_element_type=jnp.float32)
    h += jnp.dot(mean, w2a_ref[D_IN:D_IN + H, :], preferred_element_type=jnp.float32)
    h += jnp.dot(ug, w2a_ref[D_IN + H:D_IN + H + U_DIM, :],
                 preferred_element_type=jnp.float32)
    h = jax.nn.relu(h + b2a_ref[...])
    out_ref[...] = jnp.dot(h, w2b_ref[...], preferred_element_type=jnp.float32) \
        + b2b_ref[...]


def _node_mlp(x, sums, cnt, batch2d, u, W2a, b2a, W2b, b2b):
    grid = (N // BN,)
    return pl.pallas_call(
        _node_mlp_body,
        grid=grid,
        in_specs=[
            pl.BlockSpec((BN, D_IN), lambda i: (i, 0)),
            pl.BlockSpec((NC, BN, H), lambda i: (0, i, 0)),
            pl.BlockSpec((NC, BN, 1), lambda i: (0, i, 0)),
            pl.BlockSpec((BN, 1), lambda i: (i, 0)),
            pl.BlockSpec((G, U_DIM), lambda i: (0, 0)),
            pl.BlockSpec((D_IN + H + U_DIM, H), lambda i: (0, 0)),
            pl.BlockSpec((1, H), lambda i: (0, 0)),
            pl.BlockSpec((H, D_OUT), lambda i: (0, 0)),
            pl.BlockSpec((1, D_OUT), lambda i: (0, 0)),
        ],
        out_specs=pl.BlockSpec((BN, D_OUT), lambda i: (i, 0)),
        out_shape=jax.ShapeDtypeStruct((N, D_OUT), jnp.float32),
        compiler_params=pltpu.CompilerParams(
            dimension_semantics=("arbitrary",)),
    )(x, sums, cnt, batch2d, u, W2a, b2a, W2b, b2b)


def kernel(x, edge_index, edge_attr, u, batch, W1a, b1a, W1b, b1b,
           W2a, b2a, W2b, b2b):
    row = edge_index[0]
    col = edge_index[1]
    pad = EPAD - E
    col_p = jnp.concatenate([col, jnp.zeros((pad,), jnp.int32)]).reshape(EPAD // 128, 128)
    row_p = jnp.concatenate([row, jnp.full((pad,), TRASH, jnp.int32)]).reshape(EPAD // 128, 128)

    b1a2, b1b2 = b1a.reshape(1, H), b1b.reshape(1, H)
    ea_t = edge_attr.T
    g1 = _sc_gather(x, col_p[:EPAD2 // 128])
    g2 = _sc_gather(x, col_p[EPAD2 // 128:])
    msg1 = _edge_mlp(g1, ea_t, 0, EPAD2 // BE, W1a, b1a2, W1b, b1b2)
    msg2 = _edge_mlp(g2, ea_t, EPAD2 // BE, (E - EPAD2) // BE,
                     W1a, b1a2, W1b, b1b2)

    zrows = jnp.zeros((NACC, H), jnp.float32)
    zcnt = jnp.zeros((NACC,), jnp.float32)
    ones = jnp.ones((128,), jnp.float32)
    sums, cnt = _sc_scatter(msg1, msg2, row_p, zrows, zcnt, ones)

    out = _node_mlp(x, sums, cnt.reshape(NC, NACC, 1), batch.reshape(N, 1), u,
                    W2a, b2a.reshape(1, H), W2b, b2b.reshape(1, D_OUT))
    return out


# speedup vs baseline: 2.9797x; 1.1080x over previous
import functools

import jax
import jax.numpy as jnp
from jax import lax
from jax.experimental import pallas as pl
from jax.experimental.pallas import tpu as pltpu
from jax.experimental.pallas import tpu_sc as plsc

N = 10000
E = 320000
D_IN = 128
D_EDGE = 16
H = 128
D_OUT = 128
U_DIM = 64
G = 16

NC, NS = 2, 16
NW = NC * NS
EPW = 10240
EPAD = NW * EPW
GPW = EPW // 128
NCHUNK = 2
EPAD2 = EPAD // NCHUNK
EPW2 = EPW // NCHUNK
GPW2 = GPW // NCHUNK
SS = 128
NSP2 = EPW2 // SS // 2
NACC = 10240
TRASH = N
RPT = NACC // NS

_sc_mesh = plsc.VectorSubcoreMesh(core_axis_name="c", subcore_axis_name="s",
                                  num_cores=NC, num_subcores=NS)


@functools.partial(
    pl.kernel, mesh=_sc_mesh,
    out_type=jax.ShapeDtypeStruct((EPAD2, D_IN), jnp.float32),
    scratch_types=[
        pltpu.VMEM((GPW2, 128), jnp.int32),
        pltpu.VMEM((128, D_IN), jnp.float32),
        pltpu.VMEM((128, D_IN), jnp.float32),
        pltpu.SemaphoreType.DMA,
        pltpu.SemaphoreType.DMA,
        pltpu.VMEM_SHARED((NACC, D_IN), jnp.float32),
    ],
)
def _sc_gather(x_hbm, col_hbm, out_hbm, idx_v, buf0, buf1, sem0, sem1, xs_sh):
    s = lax.axis_index("s")
    wid = s * NC + lax.axis_index("c")

    @pl.when(s < NS - 1)
    def _():
        pltpu.sync_copy(x_hbm.at[pl.ds(s * RPT, RPT)], xs_sh.at[pl.ds(s * RPT, RPT)])

    @pl.when(s == NS - 1)
    def _():
        pltpu.sync_copy(x_hbm.at[pl.ds((NS - 1) * RPT, N - (NS - 1) * RPT)],
                        xs_sh.at[pl.ds((NS - 1) * RPT, N - (NS - 1) * RPT)])

    pltpu.sync_copy(col_hbm.at[pl.ds(wid * GPW2, GPW2)], idx_v)
    plsc.subcore_barrier()

    def fire(c, buf, sem):
        pltpu.async_copy(xs_sh.at[idx_v.at[c]], buf, sem)

    def drain(buf, sem):
        pltpu.make_async_copy(xs_sh.at[idx_v.at[0]], buf, sem).wait()

    def store(c, buf):
        pltpu.sync_copy(buf, out_hbm.at[pl.ds(wid * EPW2 + c * 128, 128)])

    fire(0, buf0, sem0)

    def pair(i, carry):
        fire(2 * i + 1, buf1, sem1)
        drain(buf0, sem0)
        store(2 * i, buf0)

        @pl.when(i < GPW2 // 2 - 1)
        def _():
            fire(2 * i + 2, buf0, sem0)

        drain(buf1, sem1)
        store(2 * i + 1, buf1)
        return carry

    lax.fori_loop(0, GPW2 // 2, pair, 0)


@functools.partial(
    pl.kernel, mesh=_sc_mesh,
    out_type=(
        jax.ShapeDtypeStruct((NC, NACC, H), jnp.float32),
        jax.ShapeDtypeStruct((NC, NACC), jnp.float32),
    ),
    scratch_types=[
        pltpu.VMEM((GPW2, 128), jnp.int32),
        pltpu.VMEM((SS, H), jnp.float32),
        pltpu.VMEM((SS, H), jnp.float32),
        pltpu.VMEM((128,), jnp.float32),
        pltpu.SemaphoreType.DMA,
        pltpu.SemaphoreType.DMA,
        pltpu.VMEM_SHARED((NACC, H), jnp.float32),
        pltpu.VMEM_SHARED((NACC,), jnp.float32),
    ],
)
def _sc_scatter(msg_hbm, row_hbm, zrows_hbm, zcnt_hbm, ones_hbm,
                sums_hbm, cnt_hbm, idx_v, buf0, buf1, ones_v, sem0, sem1,
                acc_sh, cacc_sh):
    c = lax.axis_index("c")
    s = lax.axis_index("s")
    wid = s * NC + c
    pltpu.sync_copy(zrows_hbm.at[pl.ds(s * RPT, RPT)], acc_sh.at[pl.ds(s * RPT, RPT)])
    pltpu.sync_copy(zcnt_hbm.at[pl.ds(s * RPT, RPT)], cacc_sh.at[pl.ds(s * RPT, RPT)])
    pltpu.sync_copy(ones_hbm, ones_v)
    pltpu.sync_copy(row_hbm.at[pl.ds(wid * GPW2, GPW2)], idx_v)
    plsc.subcore_barrier()

    base = wid * EPW2

    def fire(ch, buf, sem):
        pltpu.async_copy(msg_hbm.at[pl.ds(base + ch * SS, SS)], buf, sem)

    def drain(buf, sem):
        pltpu.make_async_copy(msg_hbm.at[pl.ds(0, SS)], buf, sem).wait()

    def scat(ch, buf):
        pltpu.sync_copy(buf, acc_sh.at[idx_v.at[ch]], add=True)
        pltpu.sync_copy(ones_v, cacc_sh.at[idx_v.at[ch]], add=True)

    fire(0, buf0, sem0)

    def pair(i, carry):
        fire(2 * i + 1, buf1, sem1)
        drain(buf0, sem0)
        scat(2 * i, buf0)

        @pl.when(i < NSP2 - 1)
        def _():
            fire(2 * i + 2, buf0, sem0)

        drain(buf1, sem1)
        scat(2 * i + 1, buf1)
        return carry

    lax.fori_loop(0, NSP2, pair, 0)
    plsc.subcore_barrier()
    pltpu.sync_copy(acc_sh.at[pl.ds(s * RPT, RPT)], sums_hbm.at[c, pl.ds(s * RPT, RPT)])
    pltpu.sync_copy(cacc_sh.at[pl.ds(s * RPT, RPT)], cnt_hbm.at[c, pl.ds(s * RPT, RPT)])


BE = 2560


def _edge_mlp_body(g_ref, ea_ref, w1a_ref, b1a_ref, w1b_ref, b1b_ref, out_ref):
    g = g_ref[...]
    ea_t = ea_ref[...]
    h = jnp.dot(g, w1a_ref[0:D_IN, :], preferred_element_type=jnp.float32)
    h += jax.lax.dot_general(
        ea_t, w1a_ref[D_IN:D_IN + D_EDGE, :], (((0,), (0,)), ((), ())),
        preferred_element_type=jnp.float32)
    h = jax.nn.relu(h + b1a_ref[...])
    h = jnp.dot(h, w1b_ref[...], preferred_element_type=jnp.float32) + b1b_ref[...]
    out_ref[...] = jax.nn.relu(h)


def _edge_mlp(gathered, ea, off, nblk, W1a, b1a, W1b, b1b):
    grid = (nblk,)
    return pl.pallas_call(
        _edge_mlp_body,
        grid=grid,
        in_specs=[
            pl.BlockSpec((BE, D_IN), lambda i: (i, 0)),
            pl.BlockSpec((D_EDGE, BE), lambda i: (0, i + off)),
            pl.BlockSpec((D_IN + D_EDGE, H), lambda i: (0, 0)),
            pl.BlockSpec((1, H), lambda i: (0, 0)),
            pl.BlockSpec((H, H), lambda i: (0, 0)),
            pl.BlockSpec((1, H), lambda i: (0, 0)),
        ],
        out_specs=pl.BlockSpec((BE, H), lambda i: (i, 0)),
        out_shape=jax.ShapeDtypeStruct((gathered.shape[0], H), jnp.float32),
        compiler_params=pltpu.CompilerParams(
            dimension_semantics=("arbitrary",)),
    )(gathered, ea, W1a, b1a, W1b, b1b)


BN = 2000


def _node_mlp_body(x_ref, sums1_ref, sums2_ref, cnt1_ref, cnt2_ref, batch_ref,
                   u_ref, w2a_ref, b2a_ref, w2b_ref, b2b_ref, out_ref):
    x = x_ref[...]
    sums = (sums1_ref[0] + sums1_ref[1]) + (sums2_ref[0] + sums2_ref[1])
    cnt = (cnt1_ref[0] + cnt1_ref[1]) + (cnt2_ref[0] + cnt2_ref[1])
    mean = sums / jnp.maximum(cnt, 1.0)
    b = batch_ref[...]
    iota_g = lax.broadcasted_iota(jnp.int32, (1, G), 1)
    onehot = (b == iota_g).astype(jnp.float32)
    ug = jnp.dot(onehot, u_ref[...], preferred_element_type=jnp.float32)
    h = jnp.dot(x, w2a_ref[0:D_IN, :], preferred_element_type=jnp.float32)
    h += jnp.dot(mean, w2a_ref[D_IN:D_IN + H, :], preferred_element_type=jnp.float32)
    h += jnp.dot(ug, w2a_ref[D_IN + H:D_IN + H + U_DIM, :],
                 preferred_element_type=jnp.float32)
    h = jax.nn.relu(h + b2a_ref[...])
    out_ref[...] = jnp.dot(h, w2b_ref[...], preferred_element_type=jnp.float32) \
        + b2b_ref[...]


def _node_mlp(x, sums1, sums2, cnt1, cnt2, batch2d, u, W2a, b2a, W2b, b2b):
    grid = (N // BN,)
    return pl.pallas_call(
        _node_mlp_body,
        grid=grid,
        in_specs=[
            pl.BlockSpec((BN, D_IN), lambda i: (i, 0)),
            pl.BlockSpec((NC, BN, H), lambda i: (0, i, 0)),
            pl.BlockSpec((NC, BN, H), lambda i: (0, i, 0)),
            pl.BlockSpec((NC, BN, 1), lambda i: (0, i, 0)),
            pl.BlockSpec((NC, BN, 1), lambda i: (0, i, 0)),
            pl.BlockSpec((BN, 1), lambda i: (i, 0)),
            pl.BlockSpec((G, U_DIM), lambda i: (0, 0)),
            pl.BlockSpec((D_IN + H + U_DIM, H), lambda i: (0, 0)),
            pl.BlockSpec((1, H), lambda i: (0, 0)),
            pl.BlockSpec((H, D_OUT), lambda i: (0, 0)),
            pl.BlockSpec((1, D_OUT), lambda i: (0, 0)),
        ],
        out_specs=pl.BlockSpec((BN, D_OUT), lambda i: (i, 0)),
        out_shape=jax.ShapeDtypeStruct((N, D_OUT), jnp.float32),
        compiler_params=pltpu.CompilerParams(
            dimension_semantics=("arbitrary",)),
    )(x, sums1, sums2, cnt1, cnt2, batch2d, u, W2a, b2a, W2b, b2b)


def kernel(x, edge_index, edge_attr, u, batch, W1a, b1a, W1b, b1b,
           W2a, b2a, W2b, b2b):
    row = edge_index[0]
    col = edge_index[1]
    pad = EPAD - E
    col_p = jnp.concatenate([col, jnp.zeros((pad,), jnp.int32)]).reshape(EPAD // 128, 128)
    row_p = jnp.concatenate([row, jnp.full((pad,), TRASH, jnp.int32)]).reshape(EPAD // 128, 128)

    b1a2, b1b2 = b1a.reshape(1, H), b1b.reshape(1, H)
    ea_t = edge_attr.T
    g1 = _sc_gather(x, col_p[:EPAD2 // 128])
    g2 = _sc_gather(x, col_p[EPAD2 // 128:])
    msg1 = _edge_mlp(g1, ea_t, 0, EPAD2 // BE, W1a, b1a2, W1b, b1b2)
    msg2 = _edge_mlp(g2, ea_t, EPAD2 // BE, (E - EPAD2) // BE,
                     W1a, b1a2, W1b, b1b2)

    zrows = jnp.zeros((NACC, H), jnp.float32)
    zcnt = jnp.zeros((NACC,), jnp.float32)
    ones = jnp.ones((128,), jnp.float32)
    sums2, cnt2 = _sc_scatter(msg2, row_p[EPAD2 // 128:], zrows, zcnt, ones)
    sums1, cnt1 = _sc_scatter(msg1, row_p[:EPAD2 // 128], zrows, zcnt, ones)

    out = _node_mlp(x, sums1, sums2, cnt1.reshape(NC, NACC, 1),
                    cnt2.reshape(NC, NACC, 1), batch.reshape(N, 1), u,
                    W2a, b2a.reshape(1, H), W2b, b2b.reshape(1, D_OUT))
    return out
